# baseline (device time: 17526 ns/iter reference)
import os

import jax
import jax.numpy as jnp
from jax import lax
from jax.experimental import pallas as pl
from jax.experimental.pallas import tpu as pltpu

M = 512
K = 512
C = 1
R = M // C


def kernel(x):
    def body(x_ref, out_ref, xbuf, sx, rx):
        my_x = lax.axis_index("x")
        my_y = lax.axis_index("y")
        px = 1 - my_x

        barrier = pltpu.get_barrier_semaphore()
        pl.semaphore_signal(
            barrier, inc=1, device_id=(px, my_y),
            device_id_type=pl.DeviceIdType.MESH,
        )
        pl.semaphore_wait(barrier, 1)

        def rdma1(c):
            rows = pl.ds(c * R, R)
            return pltpu.make_async_remote_copy(
                src_ref=x_ref.at[rows, :],
                dst_ref=xbuf.at[rows, :],
                send_sem=sx.at[c],
                recv_sem=rx.at[c],
                device_id=(px, my_y),
                device_id_type=pl.DeviceIdType.MESH,
            )

        for c in range(C):
            rdma1(c).start()

        out_ref[:, K:2 * K] = jnp.zeros((M, K), jnp.float32)
        for c in range(C):
            rows = pl.ds(c * R, R)
            rdma1(c).wait()
            out_ref[rows, 0:K] = x_ref[rows, :] + xbuf[rows, :]

    return pl.pallas_call(
        body,
        out_shape=jax.ShapeDtypeStruct((M, 2 * K), jnp.float32),
        in_specs=[pl.BlockSpec(memory_space=pltpu.VMEM)],
        out_specs=pl.BlockSpec(memory_space=pltpu.VMEM),
        scratch_shapes=[
            pltpu.VMEM((M, K), jnp.float32),
            pltpu.SemaphoreType.DMA((C,)),
            pltpu.SemaphoreType.DMA((C,)),
        ],
        compiler_params=pltpu.CompilerParams(collective_id=0),
    )(x)
